# Initial kernel scaffold; baseline (speedup 1.0000x reference)
import numpy as np
import jax
import jax.numpy as jnp
from jax import lax
from jax.experimental import pallas as pl
from jax.experimental.pallas import tpu as pltpu
from jax.experimental.pallas import tpu_sc as plsc

_NG = 256
_M = 128
_G = 32
_NF = _G * (_G - 1) // 2
_NW = 32
_CHUNK = 1568
_ROWS = _NG + 1
_BINS = _ROWS * _M
_LANES = 16


def _contrastive_T(n_gen: int) -> np.ndarray:
    cols = n_gen * (n_gen - 1) // 2
    mt = np.zeros((cols, n_gen), dtype=np.float32)
    k = 0
    for i in range(n_gen):
        for j in range(i + 1, n_gen):
            mt[k, i] = 1.0
            mt[k, j] = -1.0
            k += 1
    return mt


_CMAT_T = _contrastive_T(_G)
_CMAT = np.ascontiguousarray(_CMAT_T.T)


def _hist_body(x_hbm, b_hbm, out_hbm, xv, bv, hv):
    wid = lax.axis_index("s") * 2 + lax.axis_index("c")
    base = wid * _CHUNK
    pltpu.sync_copy(x_hbm.at[pl.ds(base, _CHUNK)], xv)
    pltpu.sync_copy(b_hbm.at[pl.ds(base, _CHUNK)], bv)

    zeros = jnp.zeros((_LANES,), jnp.float32)

    def zero_body(i, carry):
        hv[pl.ds(i * _LANES, _LANES)] = zeros
        return carry

    lax.fori_loop(0, _BINS // _LANES, zero_body, 0)

    ones = jnp.ones((_LANES,), jnp.float32)

    def acc_body(i, carry):
        xs = xv[pl.ds(i * _LANES, _LANES)]
        bs = bv[pl.ds(i * _LANES, _LANES)]
        plsc.addupdate_scatter(hv, [bs * _M + xs], ones)
        return carry

    lax.fori_loop(0, _CHUNK // _LANES, acc_body, 0)

    pltpu.sync_copy(hv, out_hbm.at[wid])


def _histogram(x_pad, b_pad):
    mesh = plsc.VectorSubcoreMesh(
        core_axis_name="c", subcore_axis_name="s", num_cores=2, num_subcores=16
    )
    k = pl.kernel(
        _hist_body,
        out_type=jax.ShapeDtypeStruct((_NW, _BINS), jnp.float32),
        mesh=mesh,
        scratch_types=[
            pltpu.VMEM((_CHUNK,), jnp.int32),
            pltpu.VMEM((_CHUNK,), jnp.int32),
            pltpu.VMEM((_BINS,), jnp.float32),
        ],
    )
    return k(x_pad, b_pad)


def _dense_body(hp_ref, prior_ref, em_ref, emt_ref, cmt_ref, cm_ref,
                ghw_ref, ghb_ref, gow_ref, gob_ref, owt_ref, ob_ref,
                n_ref, out_ref):
    f32 = jnp.float32
    hi = lax.Precision.HIGHEST
    n_nodes = n_ref[0, 0]

    hs = jnp.sum(hp_ref[...], axis=0)
    h = hs[:_NG, :]
    cnt_row = jnp.sum(h, axis=0, keepdims=True)

    lp = prior_ref[...]
    mx = jnp.max(lp, axis=1, keepdims=True)
    lp = lp - (jnp.log(jnp.sum(jnp.exp(lp - mx), axis=1, keepdims=True)) + mx)

    emt = emt_ref[...]
    mx = jnp.max(emt, axis=0, keepdims=True)
    emt = emt - (jnp.log(jnp.sum(jnp.exp(emt - mx), axis=0, keepdims=True)) + mx)
    t = emt + lp[None, :, :]
    mx = jnp.max(t, axis=2, keepdims=True)
    ll_mg = jnp.log(jnp.sum(jnp.exp(t - mx), axis=2)) + mx[:, :, 0]

    em = em_ref[...]
    mx = jnp.max(em, axis=2, keepdims=True)
    em = em - (jnp.log(jnp.sum(jnp.exp(em - mx), axis=2, keepdims=True)) + mx)
    t2 = em + lp[:, :, None]
    mx = jnp.max(t2, axis=1, keepdims=True)
    ll_gm = jnp.log(jnp.sum(jnp.exp(t2 - mx), axis=1)) + mx[:, 0, :]

    inv_n = 1.0 / n_nodes
    mean_row = jnp.dot(cnt_row, ll_mg, precision=hi,
                       preferred_element_type=f32) * inv_n
    e2_row = jnp.dot(cnt_row, ll_mg * ll_mg, precision=hi,
                     preferred_element_type=f32) * inv_n
    var_row = e2_row - mean_row * mean_row
    inv_row = lax.rsqrt(var_row + 1e-5)

    bn_mg = (ll_mg - mean_row) * inv_row
    c_tab = jnp.tanh(jnp.dot(bn_mg, cm_ref[...], precision=hi,
                             preferred_element_type=f32))

    cmt_s = cmt_ref[...] * inv_row
    off_col = jnp.sum(cmt_s * mean_row, axis=1, keepdims=True)
    c_tab_t = jnp.tanh(jnp.dot(cmt_s, ll_gm, precision=hi,
                               preferred_element_type=f32) - off_col)

    g1t = jnp.tanh(jnp.dot(ghw_ref[...], c_tab_t, precision=hi,
                           preferred_element_type=f32) + ghb_ref[...])
    gate_row = (jnp.sum(g1t * gow_ref[...], axis=0, keepdims=True)
                + gob_ref[...])

    mask = h > 0.0
    gmax = jnp.max(jnp.where(mask, gate_row, -1e30), axis=1, keepdims=True)
    e = jnp.exp(jnp.minimum(gate_row - gmax, 0.0))
    w = h * e
    denom = jnp.sum(w, axis=1, keepdims=True) + 1e-16
    wn = w / denom

    r = jnp.dot(wn, c_tab, precision=hi, preferred_element_type=f32)
    out_ref[...] = (jnp.dot(r, owt_ref[...], precision=hi,
                            preferred_element_type=f32) + ob_ref[...])


def kernel(x, edge_index, batch, prior_logits, emission_logits,
           gate_h_w, gate_h_b, gate_out_w, gate_out_b, out_w, out_b):
    del edge_index
    n = x.shape[0]
    pad = _NW * _CHUNK - n
    x_pad = jnp.concatenate([x.astype(jnp.int32),
                             jnp.zeros((pad,), jnp.int32)])
    b_pad = jnp.concatenate([batch.astype(jnp.int32),
                             jnp.full((pad,), _NG, jnp.int32)])

    hpart = _histogram(x_pad, b_pad).reshape(_NW, _ROWS, _M)

    em = emission_logits.astype(jnp.float32)
    emt = jnp.transpose(em, (2, 0, 1))
    n_arr = jnp.full((1, 1), float(n), jnp.float32)

    out = pl.pallas_call(
        _dense_body,
        out_shape=jax.ShapeDtypeStruct((_NG, 10), jnp.float32),
    )(
        hpart,
        prior_logits.astype(jnp.float32),
        em,
        emt,
        jnp.asarray(_CMAT_T),
        jnp.asarray(_CMAT),
        gate_h_w.astype(jnp.float32),
        gate_h_b.astype(jnp.float32).reshape(128, 1),
        gate_out_w.astype(jnp.float32).reshape(128, 1),
        gate_out_b.astype(jnp.float32).reshape(1, 1),
        jnp.transpose(out_w.astype(jnp.float32)),
        out_b.astype(jnp.float32).reshape(1, 10),
        n_arr,
    )
    return out


# speedup vs baseline: 54.5397x; 54.5397x over previous
import numpy as np
import jax
import jax.numpy as jnp
from jax import lax
from jax.experimental import pallas as pl
from jax.experimental.pallas import tpu as pltpu
from jax.experimental.pallas import tpu_sc as plsc

_NG = 256
_M = 128
_G = 32
_NF = _G * (_G - 1) // 2
_NW = 32
_CHUNK = 1568
_ROWS = _NG + 1
_BINS = _ROWS * _M
_LANES = 16


def _contrastive_T(n_gen: int) -> np.ndarray:
    cols = n_gen * (n_gen - 1) // 2
    mt = np.zeros((cols, n_gen), dtype=np.float32)
    k = 0
    for i in range(n_gen):
        for j in range(i + 1, n_gen):
            mt[k, i] = 1.0
            mt[k, j] = -1.0
            k += 1
    return mt


_CMAT_T = _contrastive_T(_G)
_CMAT = np.ascontiguousarray(_CMAT_T.T)


def _hist_body(x_hbm, b_hbm, out_hbm, xv, bv, hv):
    wid = lax.axis_index("s") * 2 + lax.axis_index("c")
    base = wid * _CHUNK
    pltpu.sync_copy(x_hbm.at[pl.ds(base, _CHUNK)], xv)
    pltpu.sync_copy(b_hbm.at[pl.ds(base, _CHUNK)], bv)

    zeros = jnp.zeros((_LANES,), jnp.float32)

    def zero_body(i, carry):
        hv[pl.ds(i * _LANES, _LANES)] = zeros
        return carry

    lax.fori_loop(0, _BINS // _LANES, zero_body, 0)

    ones = jnp.ones((_LANES,), jnp.float32)

    def acc_body(i, carry):
        xs = xv[pl.ds(i * _LANES, _LANES)]
        bs = bv[pl.ds(i * _LANES, _LANES)]
        plsc.addupdate_scatter(hv, [bs * _M + xs], ones)
        return carry

    lax.fori_loop(0, _CHUNK // _LANES, acc_body, 0)

    pltpu.sync_copy(hv, out_hbm.at[wid])


def _histogram(x_pad, b_pad):
    mesh = plsc.VectorSubcoreMesh(
        core_axis_name="c", subcore_axis_name="s", num_cores=2, num_subcores=16
    )
    k = pl.kernel(
        _hist_body,
        out_type=jax.ShapeDtypeStruct((_NW, _BINS), jnp.float32),
        mesh=mesh,
        scratch_types=[
            pltpu.VMEM((_CHUNK,), jnp.int32),
            pltpu.VMEM((_CHUNK,), jnp.int32),
            pltpu.VMEM((_BINS,), jnp.float32),
        ],
        compiler_params=pltpu.CompilerParams(needs_layout_passes=False),
    )
    return k(x_pad, b_pad)


def _dense_body(hp_ref, prior_ref, em_ref, emt_ref, cmt_ref, cm_ref,
                ghw_ref, ghb_ref, gow_ref, gob_ref, owt_ref, ob_ref,
                n_ref, out_ref):
    f32 = jnp.float32
    hi = lax.Precision.HIGHEST
    n_nodes = n_ref[0, 0]

    hs = jnp.sum(hp_ref[...], axis=0)
    h = hs[:_NG, :]
    cnt_row = jnp.sum(h, axis=0, keepdims=True)

    lp = prior_ref[...]
    mx = jnp.max(lp, axis=1, keepdims=True)
    lp = lp - (jnp.log(jnp.sum(jnp.exp(lp - mx), axis=1, keepdims=True)) + mx)

    emt = emt_ref[...]
    mx = jnp.max(emt, axis=0, keepdims=True)
    emt = emt - (jnp.log(jnp.sum(jnp.exp(emt - mx), axis=0, keepdims=True)) + mx)
    t = emt + lp[None, :, :]
    mx = jnp.max(t, axis=2, keepdims=True)
    ll_mg = jnp.log(jnp.sum(jnp.exp(t - mx), axis=2)) + mx[:, :, 0]

    em = em_ref[...]
    mx = jnp.max(em, axis=2, keepdims=True)
    em = em - (jnp.log(jnp.sum(jnp.exp(em - mx), axis=2, keepdims=True)) + mx)
    t2 = em + lp[:, :, None]
    mx = jnp.max(t2, axis=1, keepdims=True)
    ll_gm = jnp.log(jnp.sum(jnp.exp(t2 - mx), axis=1)) + mx[:, 0, :]

    inv_n = 1.0 / n_nodes
    mean_row = jnp.dot(cnt_row, ll_mg, precision=hi,
                       preferred_element_type=f32) * inv_n
    e2_row = jnp.dot(cnt_row, ll_mg * ll_mg, precision=hi,
                     preferred_element_type=f32) * inv_n
    var_row = e2_row - mean_row * mean_row
    inv_row = lax.rsqrt(var_row + 1e-5)

    bn_mg = (ll_mg - mean_row) * inv_row
    c_tab = jnp.tanh(jnp.dot(bn_mg, cm_ref[...], precision=hi,
                             preferred_element_type=f32))

    cmt_s = cmt_ref[...] * inv_row
    off_col = jnp.sum(cmt_s * mean_row, axis=1, keepdims=True)
    c_tab_t = jnp.tanh(jnp.dot(cmt_s, ll_gm, precision=hi,
                               preferred_element_type=f32) - off_col)

    g1t = jnp.tanh(jnp.dot(ghw_ref[...], c_tab_t, precision=hi,
                           preferred_element_type=f32) + ghb_ref[...])
    gate_row = (jnp.sum(g1t * gow_ref[...], axis=0, keepdims=True)
                + gob_ref[...])

    mask = h > 0.0
    gmax = jnp.max(jnp.where(mask, gate_row, -1e30), axis=1, keepdims=True)
    e = jnp.exp(jnp.minimum(gate_row - gmax, 0.0))
    w = h * e
    denom = jnp.sum(w, axis=1, keepdims=True) + 1e-16
    wn = w / denom

    r = jnp.dot(wn, c_tab, precision=hi, preferred_element_type=f32)
    out_ref[...] = (jnp.dot(r, owt_ref[...], precision=hi,
                            preferred_element_type=f32) + ob_ref[...])


def kernel(x, edge_index, batch, prior_logits, emission_logits,
           gate_h_w, gate_h_b, gate_out_w, gate_out_b, out_w, out_b):
    del edge_index
    n = x.shape[0]
    pad = _NW * _CHUNK - n
    x_pad = jnp.concatenate([x.astype(jnp.int32),
                             jnp.zeros((pad,), jnp.int32)])
    b_pad = jnp.concatenate([batch.astype(jnp.int32),
                             jnp.full((pad,), _NG, jnp.int32)])

    hpart = _histogram(x_pad, b_pad).reshape(_NW, _ROWS, _M)

    em = emission_logits.astype(jnp.float32)
    emt = jnp.transpose(em, (2, 0, 1))
    n_arr = jnp.full((1, 1), float(n), jnp.float32)

    out = pl.pallas_call(
        _dense_body,
        out_shape=jax.ShapeDtypeStruct((_NG, 10), jnp.float32),
    )(
        hpart,
        prior_logits.astype(jnp.float32),
        em,
        emt,
        jnp.asarray(_CMAT_T),
        jnp.asarray(_CMAT),
        gate_h_w.astype(jnp.float32),
        gate_h_b.astype(jnp.float32).reshape(128, 1),
        gate_out_w.astype(jnp.float32).reshape(128, 1),
        gate_out_b.astype(jnp.float32).reshape(1, 1),
        jnp.transpose(out_w.astype(jnp.float32)),
        out_b.astype(jnp.float32).reshape(1, 10),
        n_arr,
    )
    return out


# speedup vs baseline: 59.4441x; 1.0899x over previous
import numpy as np
import jax
import jax.numpy as jnp
from jax import lax
from jax.experimental import pallas as pl
from jax.experimental.pallas import tpu as pltpu
from jax.experimental.pallas import tpu_sc as plsc

_NG = 256
_M = 128
_G = 32
_NF = _G * (_G - 1) // 2
_NW = 32
_CHUNK = 1568
_ROWS = _NG + 1
_BINS = _ROWS * _M
_LANES = 16


def _contrastive_T(n_gen: int) -> np.ndarray:
    cols = n_gen * (n_gen - 1) // 2
    mt = np.zeros((cols, n_gen), dtype=np.float32)
    k = 0
    for i in range(n_gen):
        for j in range(i + 1, n_gen):
            mt[k, i] = 1.0
            mt[k, j] = -1.0
            k += 1
    return mt


_CMAT_T = _contrastive_T(_G)
_CMAT = np.ascontiguousarray(_CMAT_T.T)


def _hist_body(x_hbm, b_hbm, out_hbm, xv, bv, hv, xsem, bsem):
    wid = lax.axis_index("s") * 2 + lax.axis_index("c")
    base = wid * _CHUNK
    xcp = pltpu.make_async_copy(x_hbm.at[pl.ds(base, _CHUNK)], xv, xsem)
    bcp = pltpu.make_async_copy(b_hbm.at[pl.ds(base, _CHUNK)], bv, bsem)
    xcp.start()
    bcp.start()

    zeros = jnp.zeros((_LANES,), jnp.float32)

    @plsc.parallel_loop(0, _BINS // _LANES, unroll=8)
    def _zero(i):
        hv[pl.ds(i * _LANES, _LANES)] = zeros

    xcp.wait()
    bcp.wait()

    ones = jnp.ones((_LANES,), jnp.float32)

    @plsc.parallel_loop(0, _CHUNK // _LANES, unroll=4)
    def _acc(i):
        xs = xv[pl.ds(i * _LANES, _LANES)]
        bs = bv[pl.ds(i * _LANES, _LANES)]
        plsc.addupdate_scatter(hv, [bs * _M + xs], ones)

    pltpu.sync_copy(hv, out_hbm.at[wid])


def _histogram(x_pad, b_pad):
    mesh = plsc.VectorSubcoreMesh(
        core_axis_name="c", subcore_axis_name="s", num_cores=2, num_subcores=16
    )
    k = pl.kernel(
        _hist_body,
        out_type=jax.ShapeDtypeStruct((_NW, _BINS), jnp.float32),
        mesh=mesh,
        scratch_types=[
            pltpu.VMEM((_CHUNK,), jnp.int32),
            pltpu.VMEM((_CHUNK,), jnp.int32),
            pltpu.VMEM((_BINS,), jnp.float32),
            pltpu.SemaphoreType.DMA,
            pltpu.SemaphoreType.DMA,
        ],
        compiler_params=pltpu.CompilerParams(needs_layout_passes=False),
    )
    return k(x_pad, b_pad)


def _dense_body(hp_ref, prior_ref, em_ref, emt_ref, cmt_ref, cm_ref,
                ghw_ref, ghb_ref, gow_ref, gob_ref, owt_ref, ob_ref,
                n_ref, out_ref):
    f32 = jnp.float32
    hi = lax.Precision.HIGHEST
    n_nodes = n_ref[0, 0]

    hs = jnp.sum(hp_ref[...], axis=0)
    h = hs[:_NG, :]
    cnt_row = jnp.sum(h, axis=0, keepdims=True)

    lp = prior_ref[...]
    mx = jnp.max(lp, axis=1, keepdims=True)
    lp = lp - (jnp.log(jnp.sum(jnp.exp(lp - mx), axis=1, keepdims=True)) + mx)

    emt = emt_ref[...]
    mx = jnp.max(emt, axis=0, keepdims=True)
    emt = emt - (jnp.log(jnp.sum(jnp.exp(emt - mx), axis=0, keepdims=True)) + mx)
    t = emt + lp[None, :, :]
    mx = jnp.max(t, axis=2, keepdims=True)
    ll_mg = jnp.log(jnp.sum(jnp.exp(t - mx), axis=2)) + mx[:, :, 0]

    em = em_ref[...]
    mx = jnp.max(em, axis=2, keepdims=True)
    em = em - (jnp.log(jnp.sum(jnp.exp(em - mx), axis=2, keepdims=True)) + mx)
    t2 = em + lp[:, :, None]
    mx = jnp.max(t2, axis=1, keepdims=True)
    ll_gm = jnp.log(jnp.sum(jnp.exp(t2 - mx), axis=1)) + mx[:, 0, :]

    inv_n = 1.0 / n_nodes
    mean_row = jnp.dot(cnt_row, ll_mg, precision=hi,
                       preferred_element_type=f32) * inv_n
    e2_row = jnp.dot(cnt_row, ll_mg * ll_mg, precision=hi,
                     preferred_element_type=f32) * inv_n
    var_row = e2_row - mean_row * mean_row
    inv_row = lax.rsqrt(var_row + 1e-5)

    bn_mg = (ll_mg - mean_row) * inv_row
    c_tab = jnp.tanh(jnp.dot(bn_mg, cm_ref[...], precision=hi,
                             preferred_element_type=f32))

    cmt_s = cmt_ref[...] * inv_row
    off_col = jnp.sum(cmt_s * mean_row, axis=1, keepdims=True)
    c_tab_t = jnp.tanh(jnp.dot(cmt_s, ll_gm, precision=hi,
                               preferred_element_type=f32) - off_col)

    g1t = jnp.tanh(jnp.dot(ghw_ref[...], c_tab_t, precision=hi,
                           preferred_element_type=f32) + ghb_ref[...])
    gate_row = (jnp.sum(g1t * gow_ref[...], axis=0, keepdims=True)
                + gob_ref[...])

    mask = h > 0.0
    gmax = jnp.max(jnp.where(mask, gate_row, -1e30), axis=1, keepdims=True)
    e = jnp.exp(jnp.minimum(gate_row - gmax, 0.0))
    w = h * e
    denom = jnp.sum(w, axis=1, keepdims=True) + 1e-16
    wn = w / denom

    r = jnp.dot(wn, c_tab, precision=hi, preferred_element_type=f32)
    out_ref[...] = (jnp.dot(r, owt_ref[...], precision=hi,
                            preferred_element_type=f32) + ob_ref[...])


def kernel(x, edge_index, batch, prior_logits, emission_logits,
           gate_h_w, gate_h_b, gate_out_w, gate_out_b, out_w, out_b):
    del edge_index
    n = x.shape[0]
    pad = _NW * _CHUNK - n
    x_pad = jnp.concatenate([x.astype(jnp.int32),
                             jnp.zeros((pad,), jnp.int32)])
    b_pad = jnp.concatenate([batch.astype(jnp.int32),
                             jnp.full((pad,), _NG, jnp.int32)])

    hpart = _histogram(x_pad, b_pad).reshape(_NW, _ROWS, _M)

    em = emission_logits.astype(jnp.float32)
    emt = jnp.transpose(em, (2, 0, 1))
    n_arr = jnp.full((1, 1), float(n), jnp.float32)

    out = pl.pallas_call(
        _dense_body,
        out_shape=jax.ShapeDtypeStruct((_NG, 10), jnp.float32),
    )(
        hpart,
        prior_logits.astype(jnp.float32),
        em,
        emt,
        jnp.asarray(_CMAT_T),
        jnp.asarray(_CMAT),
        gate_h_w.astype(jnp.float32),
        gate_h_b.astype(jnp.float32).reshape(128, 1),
        gate_out_w.astype(jnp.float32).reshape(128, 1),
        gate_out_b.astype(jnp.float32).reshape(1, 1),
        jnp.transpose(out_w.astype(jnp.float32)),
        out_b.astype(jnp.float32).reshape(1, 10),
        n_arr,
    )
    return out


# speedup vs baseline: 74.2890x; 1.2497x over previous
import numpy as np
import jax
import jax.numpy as jnp
from jax import lax
from jax.experimental import pallas as pl
from jax.experimental.pallas import tpu as pltpu
from jax.experimental.pallas import tpu_sc as plsc

_NG = 256
_M = 128
_G = 32
_NW = 32
_N = 50000
_CHUNK = 1568
_BINS = _NG * _M
_LANES = 16
_CHUNKV = _CHUNK // _LANES
_SKIPV = (_NW * _CHUNK - _N) // _LANES // 1
_SKIPV = (_CHUNK - (_N - (_NW - 1) * _CHUNK)) // _LANES
_LAST_BASE = _N - _CHUNK

assert (_N - (_NW - 1) * _CHUNK) % _LANES == 0
assert _LAST_BASE % 8 == 0


def _contrastive_T(n_gen: int) -> np.ndarray:
    cols = n_gen * (n_gen - 1) // 2
    mt = np.zeros((cols, n_gen), dtype=np.float32)
    k = 0
    for i in range(n_gen):
        for j in range(i + 1, n_gen):
            mt[k, i] = 1.0
            mt[k, j] = -1.0
            k += 1
    return mt


_CMAT_T = _contrastive_T(_G)
_CMAT = np.ascontiguousarray(_CMAT_T.T)


def _hist_body(x_hbm, b_hbm, out_hbm, xv, bv, hv, xsem, bsem):
    wid = lax.axis_index("s") * 2 + lax.axis_index("c")
    is_last = wid == _NW - 1
    base = jnp.where(is_last, _LAST_BASE, wid * _CHUNK)
    xcp = pltpu.make_async_copy(x_hbm.at[pl.ds(base, _CHUNK)], xv, xsem)
    bcp = pltpu.make_async_copy(b_hbm.at[pl.ds(base, _CHUNK)], bv, bsem)
    xcp.start()
    bcp.start()

    zeros = jnp.zeros((_LANES,), jnp.float32)

    @plsc.parallel_loop(0, _BINS // _LANES, unroll=8)
    def _zero(i):
        hv[pl.ds(i * _LANES, _LANES)] = zeros

    xcp.wait()
    bcp.wait()

    ones = jnp.ones((_LANES,), jnp.float32)

    def _acc(i):
        xs = xv[pl.ds(i * _LANES, _LANES)]
        bs = bv[pl.ds(i * _LANES, _LANES)]
        plsc.addupdate_scatter(hv, [bs * _M + xs], ones)

    plsc.parallel_loop(_SKIPV, _CHUNKV, unroll=4)(_acc)

    @pl.when(jnp.logical_not(is_last))
    def _():
        plsc.parallel_loop(0, _SKIPV, unroll=4)(_acc)

    pltpu.sync_copy(hv, out_hbm.at[wid])


def _histogram(x, batch):
    mesh = plsc.VectorSubcoreMesh(
        core_axis_name="c", subcore_axis_name="s", num_cores=2, num_subcores=16
    )
    k = pl.kernel(
        _hist_body,
        out_type=jax.ShapeDtypeStruct((_NW, _BINS), jnp.float32),
        mesh=mesh,
        scratch_types=[
            pltpu.VMEM((_CHUNK,), jnp.int32),
            pltpu.VMEM((_CHUNK,), jnp.int32),
            pltpu.VMEM((_BINS,), jnp.float32),
            pltpu.SemaphoreType.DMA,
            pltpu.SemaphoreType.DMA,
        ],
        compiler_params=pltpu.CompilerParams(needs_layout_passes=False),
    )
    return k(x, batch)


def _dense_body(hp_ref, prior_ref, em_ref, cmt_ref, cm_ref,
                ghw_ref, ghb_ref, gow_ref, gob_ref, ow_ref, ob_ref,
                out_ref):
    f32 = jnp.float32
    hi = lax.Precision.HIGHEST

    h = jnp.sum(hp_ref[...], axis=0)
    cnt_row = jnp.sum(h, axis=0, keepdims=True)

    lp = prior_ref[...]
    mx = jnp.max(lp, axis=1, keepdims=True)
    lp = lp - (jnp.log(jnp.sum(jnp.exp(lp - mx), axis=1, keepdims=True)) + mx)

    em = em_ref[...]
    mx = jnp.max(em, axis=2, keepdims=True)
    em = em - (jnp.log(jnp.sum(jnp.exp(em - mx), axis=2, keepdims=True)) + mx)
    t2 = em + lp[:, :, None]
    mx = jnp.max(t2, axis=1, keepdims=True)
    ll_gm = jnp.log(jnp.sum(jnp.exp(t2 - mx), axis=1)) + mx[:, 0, :]
    ll_mg = jnp.transpose(ll_gm)

    inv_n = 1.0 / _N
    mean_row = jnp.dot(cnt_row, ll_mg, precision=hi,
                       preferred_element_type=f32) * inv_n
    e2_row = jnp.dot(cnt_row, ll_mg * ll_mg, precision=hi,
                     preferred_element_type=f32) * inv_n
    var_row = e2_row - mean_row * mean_row
    inv_row = lax.rsqrt(var_row + 1e-5)

    bn_mg = (ll_mg - mean_row) * inv_row
    c_tab = jnp.tanh(jnp.dot(bn_mg, cm_ref[...], precision=hi,
                             preferred_element_type=f32))

    cmt_s = cmt_ref[...] * inv_row
    off_col = jnp.sum(cmt_s * mean_row, axis=1, keepdims=True)
    c_tab_t = jnp.tanh(jnp.dot(cmt_s, ll_gm, precision=hi,
                               preferred_element_type=f32) - off_col)

    g1t = jnp.tanh(jnp.dot(ghw_ref[...], c_tab_t, precision=hi,
                           preferred_element_type=f32) + ghb_ref[...])
    gate_row = (jnp.sum(g1t * gow_ref[...], axis=0, keepdims=True)
                + gob_ref[...])

    mask = h > 0.0
    gmax = jnp.max(jnp.where(mask, gate_row, -1e30), axis=1, keepdims=True)
    e = jnp.exp(jnp.minimum(gate_row - gmax, 0.0))
    w = h * e
    denom = jnp.sum(w, axis=1, keepdims=True) + 1e-16
    wn = w / denom

    r = jnp.dot(wn, c_tab, precision=hi, preferred_element_type=f32)
    out_ref[...] = (lax.dot_general(r, ow_ref[...], (((1,), (1,)), ((), ())),
                                    precision=hi,
                                    preferred_element_type=f32) + ob_ref[...])


def kernel(x, edge_index, batch, prior_logits, emission_logits,
           gate_h_w, gate_h_b, gate_out_w, gate_out_b, out_w, out_b):
    del edge_index
    assert x.shape[0] == _N

    hpart = _histogram(x.astype(jnp.int32), batch.astype(jnp.int32))
    hpart = hpart.reshape(_NW, _NG, _M)

    out = pl.pallas_call(
        _dense_body,
        out_shape=jax.ShapeDtypeStruct((_NG, 10), jnp.float32),
    )(
        hpart,
        prior_logits.astype(jnp.float32),
        emission_logits.astype(jnp.float32),
        jnp.asarray(_CMAT_T),
        jnp.asarray(_CMAT),
        gate_h_w.astype(jnp.float32),
        gate_h_b.astype(jnp.float32).reshape(128, 1),
        gate_out_w.astype(jnp.float32).reshape(128, 1),
        gate_out_b.astype(jnp.float32).reshape(1, 1),
        out_w.astype(jnp.float32),
        out_b.astype(jnp.float32).reshape(1, 10),
    )
    return out


# speedup vs baseline: 85.9082x; 1.1564x over previous
import numpy as np
import jax
import jax.numpy as jnp
from jax import lax
from jax.experimental import pallas as pl
from jax.experimental.pallas import tpu as pltpu
from jax.experimental.pallas import tpu_sc as plsc

_NG = 256
_M = 128
_G = 32
_NW = 32
_N = 50000
_CHUNK = 1568
_BINS = _NG * _M
_LANES = 16
_CHUNKV = _CHUNK // _LANES
_SKIPV = (_NW * _CHUNK - _N) // _LANES // 1
_SKIPV = (_CHUNK - (_N - (_NW - 1) * _CHUNK)) // _LANES
_LAST_BASE = _N - _CHUNK

assert (_N - (_NW - 1) * _CHUNK) % _LANES == 0
assert _LAST_BASE % 8 == 0


def _contrastive_T(n_gen: int) -> np.ndarray:
    cols = n_gen * (n_gen - 1) // 2
    mt = np.zeros((cols, n_gen), dtype=np.float32)
    k = 0
    for i in range(n_gen):
        for j in range(i + 1, n_gen):
            mt[k, i] = 1.0
            mt[k, j] = -1.0
            k += 1
    return mt


_CMAT_T = _contrastive_T(_G)
_CMAT = np.ascontiguousarray(_CMAT_T.T)


def _hist_body(x_hbm, b_hbm, out_hbm, xv, bv, hv, xsem, bsem):
    wid = lax.axis_index("s") * 2 + lax.axis_index("c")
    is_last = wid == _NW - 1
    base = jnp.where(is_last, _LAST_BASE, wid * _CHUNK)
    xcp = pltpu.make_async_copy(x_hbm.at[pl.ds(base, _CHUNK)], xv, xsem)
    bcp = pltpu.make_async_copy(b_hbm.at[pl.ds(base, _CHUNK)], bv, bsem)
    xcp.start()
    bcp.start()

    zeros = jnp.zeros((_LANES,), jnp.float32)
    n_col = _M // _LANES

    @plsc.parallel_loop(0, _BINS // _LANES, unroll=8)
    def _zero(i):
        hv[i // n_col, pl.ds((i % n_col) * _LANES, _LANES)] = zeros

    xcp.wait()
    bcp.wait()

    ones = jnp.ones((_LANES,), jnp.float32)

    def _acc(i):
        xs = xv[pl.ds(i * _LANES, _LANES)]
        bs = bv[pl.ds(i * _LANES, _LANES)]
        plsc.addupdate_scatter(hv, [bs, xs], ones)

    plsc.parallel_loop(_SKIPV, _CHUNKV, unroll=4)(_acc)

    @pl.when(jnp.logical_not(is_last))
    def _():
        plsc.parallel_loop(0, _SKIPV, unroll=4)(_acc)

    pltpu.sync_copy(hv, out_hbm.at[wid])


def _histogram(x, batch):
    mesh = plsc.VectorSubcoreMesh(
        core_axis_name="c", subcore_axis_name="s", num_cores=2, num_subcores=16
    )
    k = pl.kernel(
        _hist_body,
        out_type=jax.ShapeDtypeStruct((_NW, _NG, _M), jnp.float32),
        mesh=mesh,
        scratch_types=[
            pltpu.VMEM((_CHUNK,), jnp.int32),
            pltpu.VMEM((_CHUNK,), jnp.int32),
            pltpu.VMEM((_NG, _M), jnp.float32),
            pltpu.SemaphoreType.DMA,
            pltpu.SemaphoreType.DMA,
        ],
        compiler_params=pltpu.CompilerParams(needs_layout_passes=False),
    )
    return k(x, batch)


def _dense_body(hp_ref, prior_ref, em_ref, cmt_ref, cm_ref,
                ghw_ref, ghb_ref, gow_ref, gob_ref, ow_ref, ob_ref,
                out_ref):
    f32 = jnp.float32
    hi = lax.Precision.HIGHEST

    h = jnp.sum(hp_ref[...], axis=0)
    cnt_row = jnp.sum(h, axis=0, keepdims=True)

    lp = prior_ref[...]
    mx = jnp.max(lp, axis=1, keepdims=True)
    lp = lp - (jnp.log(jnp.sum(jnp.exp(lp - mx), axis=1, keepdims=True)) + mx)

    em = em_ref[...]
    mx = jnp.max(em, axis=2, keepdims=True)
    em = em - (jnp.log(jnp.sum(jnp.exp(em - mx), axis=2, keepdims=True)) + mx)
    t2 = em + lp[:, :, None]
    mx = jnp.max(t2, axis=1, keepdims=True)
    ll_gm = jnp.log(jnp.sum(jnp.exp(t2 - mx), axis=1)) + mx[:, 0, :]
    ll_mg = jnp.transpose(ll_gm)

    inv_n = 1.0 / _N
    mean_row = jnp.dot(cnt_row, ll_mg, precision=hi,
                       preferred_element_type=f32) * inv_n
    e2_row = jnp.dot(cnt_row, ll_mg * ll_mg, precision=hi,
                     preferred_element_type=f32) * inv_n
    var_row = e2_row - mean_row * mean_row
    inv_row = lax.rsqrt(var_row + 1e-5)

    bn_mg = (ll_mg - mean_row) * inv_row
    c_tab = jnp.tanh(jnp.dot(bn_mg, cm_ref[...], precision=hi,
                             preferred_element_type=f32))

    cmt_s = cmt_ref[...] * inv_row
    off_col = jnp.sum(cmt_s * mean_row, axis=1, keepdims=True)
    c_tab_t = jnp.tanh(jnp.dot(cmt_s, ll_gm, precision=hi,
                               preferred_element_type=f32) - off_col)

    g1t = jnp.tanh(jnp.dot(ghw_ref[...], c_tab_t, precision=hi,
                           preferred_element_type=f32) + ghb_ref[...])
    gate_row = (jnp.sum(g1t * gow_ref[...], axis=0, keepdims=True)
                + gob_ref[...])

    mask = h > 0.0
    gmax = jnp.max(jnp.where(mask, gate_row, -1e30), axis=1, keepdims=True)
    e = jnp.exp(jnp.minimum(gate_row - gmax, 0.0))
    w = h * e
    denom = jnp.sum(w, axis=1, keepdims=True) + 1e-16
    wn = w / denom

    r = jnp.dot(wn, c_tab, precision=hi, preferred_element_type=f32)
    out_ref[...] = (lax.dot_general(r, ow_ref[...], (((1,), (1,)), ((), ())),
                                    precision=hi,
                                    preferred_element_type=f32) + ob_ref[...])


def kernel(x, edge_index, batch, prior_logits, emission_logits,
           gate_h_w, gate_h_b, gate_out_w, gate_out_b, out_w, out_b):
    del edge_index
    assert x.shape[0] == _N

    hpart = _histogram(x.astype(jnp.int32), batch.astype(jnp.int32))

    out = pl.pallas_call(
        _dense_body,
        out_shape=jax.ShapeDtypeStruct((_NG, 10), jnp.float32),
    )(
        hpart,
        prior_logits.astype(jnp.float32),
        emission_logits.astype(jnp.float32),
        jnp.asarray(_CMAT_T),
        jnp.asarray(_CMAT),
        gate_h_w.astype(jnp.float32),
        gate_h_b.astype(jnp.float32).reshape(128, 1),
        gate_out_w.astype(jnp.float32).reshape(128, 1),
        gate_out_b.astype(jnp.float32).reshape(1, 1),
        out_w.astype(jnp.float32),
        out_b.astype(jnp.float32).reshape(1, 10),
    )
    return out
